# baseline (speedup 1.0000x reference)
import jax
import jax.numpy as jnp
from jax import lax
from jax.experimental import pallas as pl
from jax.experimental.pallas import tpu as pltpu
from jax.experimental.pallas import tpu_sc as plsc

B = 4096
CONT = 13
NCAT = 26
VOCAB = 100000
D = 64
L = 16

NC = 2
NS = 16
NW = NC * NS
KPAIRS = CONT * D // NW
NBUF = 3
SPLITS = (7, 7, 6, 6)


def _make_body(nf, with_cont):
    cpairs = nf * D // NW
    base = CONT * D if with_cont else 0

    def body(table_hbm, catidx_hbm, cont_t_hbm, ctab_hbm, out_hbm,
             idx_v, ctab_v, bufs_v, cbufs_v, gsems, wsems, csems):
        wid = lax.axis_index("s") * NC + lax.axis_index("c")

        if with_cont:
            pltpu.sync_copy(ctab_hbm.at[pl.ds(0, CONT * D)], ctab_v)
            p0k = wid * KPAIRS
            for j in range(KPAIRS):
                p = p0k + j
                c = p // D
                d = p - c * D
                pltpu.sync_copy(cont_t_hbm.at[c], cbufs_v.at[j])
                ct = c * D + d
                g = (ct // L) * L
                lane = ct - g
                vals = ctab_v[pl.ds(g, L)]
                scal = vals.at[jnp.full((L,), lane, jnp.int32)].get(
                    mode="promise_in_bounds")
                buf = cbufs_v.at[j]

                def mul(s, _):
                    buf[pl.ds(s * L, L)] = buf[pl.ds(s * L, L)] * scal
                    return 0

                lax.fori_loop(0, B // L, mul, 0)

        p0 = wid * cpairs
        f0 = p0 // D
        pltpu.sync_copy(catidx_hbm.at[pl.ds(f0, 2), :], idx_v)

        def gather(j):
            r = j % NBUF
            p = p0 + j
            f = p // D
            d = p - f * D
            return pltpu.make_async_copy(
                table_hbm.at[f, d].at[idx_v.at[f - f0]],
                bufs_v.at[r], gsems.at[r])

        def writeback(j):
            r = j % NBUF
            return pltpu.make_async_copy(
                bufs_v.at[r], out_hbm.at[base + p0 + j], wsems.at[r])

        for j in range(NBUF):
            gather(j).start()
        for j in range(cpairs):
            gather(j).wait()
            writeback(j).start()
            writeback(j).wait()
            if j + NBUF < cpairs:
                gather(j + NBUF).start()

        if with_cont:
            def cwriteback(j):
                return pltpu.make_async_copy(
                    cbufs_v.at[j], out_hbm.at[wid * KPAIRS + j],
                    csems.at[j % NBUF])

            for j in range(KPAIRS):
                cwriteback(j).start()
            for j in range(KPAIRS):
                cwriteback(j).wait()

    return body


def _sc_kernel(table_t, catidx, cont_t, ctab_flat, nf, with_cont):
    mesh = plsc.VectorSubcoreMesh(core_axis_name="c", subcore_axis_name="s")
    rows = (CONT * D if with_cont else 0) + nf * D
    return pl.kernel(
        _make_body(nf, with_cont),
        out_type=jax.ShapeDtypeStruct((rows, B), jnp.float32),
        mesh=mesh,
        scratch_types=[
            pltpu.VMEM((2, B), jnp.int32),
            pltpu.VMEM((CONT * D,), jnp.float32),
            pltpu.VMEM((NBUF, B), jnp.float32),
            pltpu.VMEM((KPAIRS if with_cont else 1, B), jnp.float32),
            pltpu.SemaphoreType.DMA((NBUF,)),
            pltpu.SemaphoreType.DMA((NBUF,)),
            pltpu.SemaphoreType.DMA((NBUF,)),
        ],
        compiler_params=pltpu.CompilerParams(use_tc_tiling_on_sc=False),
    )(table_t, catidx, cont_t, ctab_flat)


@jax.jit
def kernel(continuous, categorical, cat_tables, cont_table):
    table_t = jnp.transpose(cat_tables, (0, 2, 1))
    cat_t = categorical.T
    cont_t = jnp.concatenate(
        [continuous.T, jnp.zeros((1, B), jnp.float32)], axis=0)
    ctab_flat = cont_table.reshape(CONT * D)
    pad = jnp.zeros((1, B), jnp.int32)
    parts = []
    f = 0
    for i, nf in enumerate(SPLITS):
        catidx = jnp.concatenate([cat_t[f:f + nf], pad], axis=0)
        parts.append(_sc_kernel(table_t[f:f + nf], catidx, cont_t,
                                ctab_flat, nf, i == 0))
        f += nf
    out_t = jnp.concatenate(parts, axis=0)
    return jnp.transpose(out_t.reshape(CONT + NCAT, D, B), (2, 0, 1))


# speedup vs baseline: 1.2003x; 1.2003x over previous
import functools

import jax
import jax.numpy as jnp
from jax import lax
from jax.experimental import pallas as pl
from jax.experimental.pallas import tpu as pltpu
from jax.experimental.pallas import tpu_sc as plsc

B = 4096
CONT = 13
NCAT = 26
VOCAB = 100000
D = 64
L = 16

NC = 2
NS = 16
NW = NC * NS
CPAIRS = NCAT * D // NW
KPAIRS = CONT * D // NW
IB = B // 128
NBUF = 3


def _sc_body(table_hbm, catidx_hbm, cont_t_hbm, ctab_hbm, out_hbm,
             idx_v, ctab_v, bufs_v, cbufs_v, gsems, wsems, csems):
    wid = lax.axis_index("s") * NC + lax.axis_index("c")

    pltpu.sync_copy(ctab_hbm.at[pl.ds(0, CONT * D)], ctab_v)
    p0k = wid * KPAIRS
    for j in range(KPAIRS):
        p = p0k + j
        c = p // D
        d = p - c * D
        pltpu.sync_copy(cont_t_hbm.at[c], cbufs_v.at[j])
        ct = c * D + d
        g = (ct // L) * L
        lane = ct - g
        vals = ctab_v[pl.ds(g, L)]
        scal = vals.at[jnp.full((L,), lane, jnp.int32)].get(
            mode="promise_in_bounds")
        buf = cbufs_v.at[j]

        def body(s, _):
            buf[pl.ds(s * L, L)] = buf[pl.ds(s * L, L)] * scal
            return 0

        lax.fori_loop(0, B // L, body, 0)

    p0 = wid * CPAIRS
    f0 = p0 // D
    pltpu.sync_copy(catidx_hbm.at[pl.ds(f0, 2), :], idx_v)

    def gather(j):
        r = j % NBUF
        p = p0 + j
        f = p // D
        d = p - f * D
        return pltpu.make_async_copy(
            table_hbm.at[f, d].at[idx_v.at[f - f0]],
            bufs_v.at[r], gsems.at[r])

    def writeback(j):
        r = j % NBUF
        return pltpu.make_async_copy(
            bufs_v.at[r], out_hbm.at[CONT * D + p0 + j], wsems.at[r])

    for j in range(NBUF):
        gather(j).start()
    for j in range(CPAIRS):
        gather(j).wait()
        writeback(j).start()
        writeback(j).wait()
        if j + NBUF < CPAIRS:
            gather(j + NBUF).start()

    def cwriteback(j):
        return pltpu.make_async_copy(
            cbufs_v.at[j], out_hbm.at[p0k + j], csems.at[j % NBUF])

    for j in range(KPAIRS):
        cwriteback(j).start()
    for j in range(KPAIRS):
        cwriteback(j).wait()


def _sc_kernel(table_t, catidx, cont_t, ctab_flat):
    mesh = plsc.VectorSubcoreMesh(core_axis_name="c", subcore_axis_name="s")
    return pl.kernel(
        _sc_body,
        out_type=jax.ShapeDtypeStruct(((CONT + NCAT) * D, B), jnp.float32),
        mesh=mesh,
        scratch_types=[
            pltpu.VMEM((2, B), jnp.int32),
            pltpu.VMEM((CONT * D,), jnp.float32),
            pltpu.VMEM((NBUF, B), jnp.float32),
            pltpu.VMEM((KPAIRS, B), jnp.float32),
            pltpu.SemaphoreType.DMA((NBUF,)),
            pltpu.SemaphoreType.DMA((NBUF,)),
            pltpu.SemaphoreType.DMA((NBUF,)),
        ],
        compiler_params=pltpu.CompilerParams(use_tc_tiling_on_sc=False),
    )(table_t, catidx, cont_t, ctab_flat)


@jax.jit
def kernel(continuous, categorical, cat_tables, cont_table):
    table_t = jnp.transpose(cat_tables, (0, 2, 1))
    catidx = jnp.concatenate(
        [categorical.T, jnp.zeros((1, B), jnp.int32)], axis=0)
    cont_t = jnp.concatenate(
        [continuous.T, jnp.zeros((1, B), jnp.float32)], axis=0)
    ctab_flat = cont_table.reshape(CONT * D)
    out_t = _sc_kernel(table_t, catidx, cont_t, ctab_flat)
    return jnp.transpose(out_t.reshape(CONT + NCAT, D, B), (2, 0, 1))
